# baseline (device time: 11470 ns/iter reference)
import jax
import jax.numpy as jnp
from jax import lax
from jax.experimental import pallas as pl
from jax.experimental.pallas import tpu as pltpu

N = 8
PAD = 80
ROWS = 512
NPAD = N * PAD


def _a2av_body(x_ref, dcol_ref, drow_ref, out_ref, stage_ref, dall_ref,
               send_ref, send_x, recv_x, send_d, recv_d):
    me = lax.axis_index("i")

    bsem = pltpu.get_barrier_semaphore()
    for k in range(1, N):
        pl.semaphore_signal(
            bsem, inc=1,
            device_id=((me + k) % N,),
            device_id_type=pl.DeviceIdType.MESH,
        )

    dest_col = dcol_ref[...]
    d8_row = lax.broadcasted_iota(jnp.int32, (ROWS, N), 1)
    onehot_d = (dest_col == d8_row).astype(jnp.int32)

    def _shift_down(a, s):
        return jnp.concatenate(
            [jnp.zeros((s, N), jnp.int32), a[: ROWS - s, :]], axis=0
        )

    acc = _shift_down(onehot_d, 1)
    s = 1
    while s < ROWS:
        acc = acc + _shift_down(acc, s)
        s *= 2
    rank_col = jnp.sum(acc * onehot_d, axis=1, keepdims=True)
    k_col = (dest_col - me) % N
    target_col = k_col * PAD + rank_col
    p_row = lax.broadcasted_iota(jnp.int32, (ROWS, NPAD), 1)
    s_t = (target_col == p_row).astype(jnp.bfloat16)
    x_bf = x_ref[...].astype(jnp.bfloat16)
    send_ref[...] = lax.dot_general(
        s_t, x_bf, (((0,), (0,)), ((), ())),
        preferred_element_type=jnp.float32,
    ).astype(jnp.bfloat16)

    pl.semaphore_wait(bsem, N - 1)

    descs = []
    for k in range(1, N):
        tgt = (me + k) % N
        slot = N - k
        rd = pltpu.make_async_remote_copy(
            src_ref=drow_ref,
            dst_ref=dall_ref.at[slot],
            send_sem=send_d.at[k - 1],
            recv_sem=recv_d.at[slot - 1],
            device_id=(tgt,),
            device_id_type=pl.DeviceIdType.MESH,
        )
        rd.start()
        rx = pltpu.make_async_remote_copy(
            src_ref=send_ref.at[pl.ds(k * PAD, PAD), :],
            dst_ref=stage_ref.at[pl.ds(slot * PAD, PAD), :],
            send_sem=send_x.at[k - 1],
            recv_sem=recv_x.at[slot - 1],
            device_id=(tgt,),
            device_id_type=pl.DeviceIdType.MESH,
        )
        rx.start()
        descs.append((rx, rd))

    stage_ref[0:PAD, :] = send_ref[0:PAD, :]
    dall_ref[0] = drow_ref[...]

    for _, rd in descs:
        rd.wait()

    dall = dall_ref[...]
    cnt_rel = jnp.sum((dall == me).astype(jnp.int32), axis=2)
    p_io = lax.broadcasted_iota(jnp.int32, (1, NPAD), 1)
    j_p = p_io // PAD
    u_p = p_io % PAD
    s_p = (me + j_p) % N
    io_r8 = lax.broadcasted_iota(jnp.int32, (N, NPAD), 0)
    cnt_of_p = jnp.sum(
        (io_r8 == j_p).astype(jnp.int32) * cnt_rel, axis=0, keepdims=True
    )
    src_of_slot = (me + io_r8) % N
    off_of_p = jnp.sum(
        (src_of_slot < s_p).astype(jnp.int32) * cnt_rel,
        axis=0, keepdims=True,
    )
    t_p = jnp.where(u_p < cnt_of_p, off_of_p + u_p, ROWS)
    t_col = lax.broadcasted_iota(jnp.int32, (ROWS, NPAD), 0)
    p_mat = (t_col == t_p).astype(jnp.bfloat16)

    for rx, _ in descs:
        rx.wait()
    out_ref[...] = jnp.dot(
        p_mat, stage_ref[...], preferred_element_type=jnp.float32
    ).astype(jnp.bfloat16)


def kernel(x, dest):
    rows, cols = x.shape
    dcol = dest.reshape(rows, 1)
    drow = dest.reshape(1, rows)

    return pl.pallas_call(
        _a2av_body,
        out_shape=jax.ShapeDtypeStruct((rows, cols), jnp.bfloat16),
        in_specs=[
            pl.BlockSpec(memory_space=pltpu.VMEM),
            pl.BlockSpec(memory_space=pltpu.VMEM),
            pl.BlockSpec(memory_space=pltpu.VMEM),
        ],
        out_specs=pl.BlockSpec(memory_space=pltpu.VMEM),
        scratch_shapes=[
            pltpu.VMEM((NPAD, cols), jnp.bfloat16),
            pltpu.VMEM((N, 1, rows), jnp.int32),
            pltpu.VMEM((NPAD, cols), jnp.bfloat16),
            pltpu.SemaphoreType.DMA((N - 1,)),
            pltpu.SemaphoreType.DMA((N - 1,)),
            pltpu.SemaphoreType.DMA((N - 1,)),
            pltpu.SemaphoreType.DMA((N - 1,)),
        ],
        compiler_params=pltpu.CompilerParams(collective_id=0),
    )(x, dcol, drow)
